# baseline (speedup 1.0000x reference)
import jax
import jax.numpy as jnp
from jax import lax
from jax.experimental import pallas as pl
from jax.experimental.pallas import tpu as pltpu
from jax.experimental.pallas import tpu_sc as plsc

N = 50000
E = 800000
D = 64
H = 32
NTILES = 16
NP = 51200
EP = 819200
CHUNK = 128
IB = 10
R = 5
G = 3
ROWS_PER_TILE = NP // NTILES
E_PER_TILE = EP // NTILES
N_CHUNKS = E_PER_TILE // CHUNK
N_BLOCKS = N_CHUNKS // IB
IB_DEG = 10
E_PER_TILE_DEG = EP // (2 * NTILES)
N_CHUNKS_DEG = E_PER_TILE_DEG // CHUNK
N_BLOCKS_DEG = N_CHUNKS_DEG // IB_DEG
B = 1024
GRID = NP // B


def _sc_aggregate(do_deg):
    mesh = plsc.VectorSubcoreMesh(core_axis_name="c", subcore_axis_name="s")

    out_type = [jax.ShapeDtypeStruct((2, NP, H), jnp.float32)]
    scratch = [
        pltpu.VMEM((IB, CHUNK), jnp.int32),
        pltpu.VMEM((IB, CHUNK), jnp.int32),
        pltpu.VMEM((R, CHUNK, H), jnp.float32),
        pltpu.VMEM_SHARED((NP, H), jnp.float32),
        pltpu.SemaphoreType.DMA,
        pltpu.SemaphoreType.DMA,
    ]
    if do_deg:
        out_type.append(jax.ShapeDtypeStruct((2, NP), jnp.float32))
        scratch += [
            pltpu.VMEM((CHUNK,), jnp.float32),
            pltpu.VMEM_SHARED((NP,), jnp.float32),
        ]

    def agg_loop(c, s, src_hbm, dst_hbm, m_hbm, srcb, dstb, rows,
                 acc, sem_g, sem_s):
        ch0 = s * N_CHUNKS
        off = c * NP

        def block(b, carry):
            blk = ch0 + b * IB
            pltpu.sync_copy(src_hbm.at[pl.ds(blk, IB)], srcb)
            pltpu.sync_copy(dst_hbm.at[pl.ds(blk, IB)], dstb)
            for j in range(IB):
                for k in range(CHUNK // 16):
                    sl = pl.ds(k * 16, 16)
                    srcb[j, sl] = srcb[j, sl] + off
            gd = {j: pltpu.async_copy(m_hbm.at[srcb.at[j]], rows.at[j],
                                      sem_g)
                  for j in range(G)}
            for j in range(IB):
                gd[j].wait()
                pltpu.sync_copy(rows.at[j % R], acc.at[dstb.at[j]],
                                add=True)
                nj = j + G
                if nj < IB:
                    gd[nj] = pltpu.async_copy(m_hbm.at[srcb.at[nj]],
                                              rows.at[nj % R], sem_g)
            return carry

        lax.fori_loop(0, N_BLOCKS, block, 0)

    def deg_loop(c, s, dst_hbm, dstb, ones_v, dacc):
        ch0 = (c * NTILES + s) * N_CHUNKS_DEG

        def block(b, carry):
            blk = ch0 + b * IB_DEG
            pltpu.sync_copy(dst_hbm.at[pl.ds(blk, IB_DEG)], dstb)
            for j in range(IB_DEG):
                pltpu.sync_copy(ones_v, dacc.at[dstb.at[j]], add=True)
            return carry

        lax.fori_loop(0, N_BLOCKS_DEG, block, 0)

    def body_deg(src_hbm, dst_hbm, m_hbm, z2_hbm, z1_hbm,
                 ones_hbm, agg_out, deg_out, srcb, dstb, rows, acc,
                 sem_g, sem_s, ones_v, dacc):
        c = lax.axis_index("c")
        s = lax.axis_index("s")
        r0 = s * ROWS_PER_TILE
        pltpu.sync_copy(z2_hbm, acc.at[pl.ds(r0, ROWS_PER_TILE)])
        pltpu.sync_copy(z1_hbm, dacc.at[pl.ds(r0, ROWS_PER_TILE)])
        pltpu.sync_copy(ones_hbm, ones_v)
        plsc.subcore_barrier()

        agg_loop(c, s, src_hbm, dst_hbm, m_hbm, srcb, dstb,
                 rows, acc, sem_g, sem_s)
        deg_loop(c, s, dst_hbm, dstb, ones_v, dacc)

        plsc.subcore_barrier()
        pltpu.sync_copy(acc.at[pl.ds(r0, ROWS_PER_TILE)],
                        agg_out.at[c, pl.ds(r0, ROWS_PER_TILE)])
        pltpu.sync_copy(dacc.at[pl.ds(r0, ROWS_PER_TILE)],
                        deg_out.at[c, pl.ds(r0, ROWS_PER_TILE)])

    def body_nodeg(src_hbm, dst_hbm, m_hbm, z2_hbm,
                   agg_out, srcb, dstb, rows, acc, sem_g, sem_s):
        c = lax.axis_index("c")
        s = lax.axis_index("s")
        r0 = s * ROWS_PER_TILE
        pltpu.sync_copy(z2_hbm, acc.at[pl.ds(r0, ROWS_PER_TILE)])
        plsc.subcore_barrier()

        agg_loop(c, s, src_hbm, dst_hbm, m_hbm, srcb, dstb,
                 rows, acc, sem_g, sem_s)

        plsc.subcore_barrier()
        pltpu.sync_copy(acc.at[pl.ds(r0, ROWS_PER_TILE)],
                        agg_out.at[c, pl.ds(r0, ROWS_PER_TILE)])

    body = body_deg if do_deg else body_nodeg
    return pl.kernel(body, out_type=out_type, mesh=mesh,
                     scratch_types=scratch,
                     compiler_params=pltpu.CompilerParams(
                         use_tc_tiling_on_sc=False))


_sc_agg_deg = _sc_aggregate(True)
_sc_agg = _sc_aggregate(False)

_W_SPEC = pl.BlockSpec((D, D), lambda i: (0, 0))
_B_SPEC = pl.BlockSpec((1, D), lambda i: (0, 0))
_ROW_SPEC = pl.BlockSpec((B, D), lambda i: (i, 0))
_HALF_SPEC = pl.BlockSpec((B, H), lambda i: (i, 0))
_AGG_SPEC = pl.BlockSpec((2, B, H), lambda i: (0, i, 0))
_DEG_SPEC = pl.BlockSpec((2, B, 1), lambda i: (0, i, 0))
_M_SHAPE = jax.ShapeDtypeStruct((NP, H), jnp.float32)
_T_SHAPE = jax.ShapeDtypeStruct((NP, D), jnp.float32)


def _tc_messages(h, Wn):
    def body(h_ref, wn0_ref, wn1_ref, m_ref):
        c = pl.program_id(1)
        wn = jnp.where(c == 0, wn0_ref[...], wn1_ref[...])
        m_ref[...] = jnp.dot(h_ref[...], wn,
                             preferred_element_type=jnp.float32)

    return pl.pallas_call(
        body, grid=(GRID, 2),
        in_specs=[
            pl.BlockSpec((B, D), lambda i, c: (i, 0)),
            pl.BlockSpec((D, H), lambda i, c: (0, 0)),
            pl.BlockSpec((D, H), lambda i, c: (0, 0)),
        ],
        out_specs=pl.BlockSpec((B, H), lambda i, c: (c * GRID + i, 0)),
        out_shape=jax.ShapeDtypeStruct((2 * NP, H), jnp.float32),
    )(h, Wn[:, :H], Wn[:, H:])


def _tc_self(h, Ws, b):
    def body(h_ref, ws_ref, b_ref, t_ref):
        t_ref[...] = jnp.dot(h_ref[...], ws_ref[...],
                             preferred_element_type=jnp.float32) + b_ref[...]

    return pl.pallas_call(
        body, grid=(GRID,),
        in_specs=[_ROW_SPEC, _W_SPEC, _B_SPEC],
        out_specs=_ROW_SPEC,
        out_shape=_T_SHAPE,
    )(h, Ws, b)


def _tc_combine(t1, agg, deg, Wn):
    def body(t_ref, a_ref, d_ref, wn0_ref, wn1_ref, h_ref, m_ref):
        c = pl.program_id(1)
        a = jnp.concatenate([a_ref[0], a_ref[1]], axis=1)
        dg = jnp.maximum(d_ref[0] + d_ref[1], 1.0)
        h1 = jnp.maximum(t_ref[...] + a / dg, 0.0)
        h_ref[...] = h1
        wn = jnp.where(c == 0, wn0_ref[...], wn1_ref[...])
        m_ref[...] = jnp.dot(h1, wn, preferred_element_type=jnp.float32)

    return pl.pallas_call(
        body, grid=(GRID, 2),
        in_specs=[
            pl.BlockSpec((B, D), lambda i, c: (i, 0)),
            pl.BlockSpec((2, B, H), lambda i, c: (0, i, 0)),
            pl.BlockSpec((2, B, 1), lambda i, c: (0, i, 0)),
            pl.BlockSpec((D, H), lambda i, c: (0, 0)),
            pl.BlockSpec((D, H), lambda i, c: (0, 0)),
        ],
        out_specs=[
            pl.BlockSpec((B, D), lambda i, c: (i, 0)),
            pl.BlockSpec((B, H), lambda i, c: (c * GRID + i, 0)),
        ],
        out_shape=[_T_SHAPE,
                   jax.ShapeDtypeStruct((2 * NP, H), jnp.float32)],
    )(t1, agg, deg, Wn[:, :H], Wn[:, H:])


def _tc_finish(t2, agg, deg, wp, bp):
    def body(t_ref, a_ref, d_ref, wp_ref, bp_ref, o_ref, mx, sm):
        i = pl.program_id(0)
        a = jnp.concatenate([a_ref[0], a_ref[1]], axis=1)
        dg = jnp.maximum(d_ref[0] + d_ref[1], 1.0)
        h2 = t_ref[...] + a / dg
        rid = i * B + lax.broadcasted_iota(jnp.int32, (B, 1), 0)
        valid = rid < N
        pmax = jnp.max(jnp.where(valid, h2, -jnp.inf), axis=0,
                       keepdims=True)
        psum = jnp.sum(jnp.where(valid, h2, 0.0), axis=0, keepdims=True)

        @pl.when(i == 0)
        def _():
            mx[...] = pmax
            sm[...] = psum

        @pl.when(i > 0)
        def _():
            mx[...] = jnp.maximum(mx[...], pmax)
            sm[...] = sm[...] + psum

        @pl.when(i == GRID - 1)
        def _():
            pooled = jnp.concatenate([mx[...], sm[...]], axis=1)
            v = (jnp.sum(pooled * wp_ref[...], axis=1, keepdims=True)
                 + bp_ref[...])
            o_ref[...] = jnp.maximum(v, 0.0) + jnp.log(
                1.0 + jnp.exp(-jnp.abs(v)))

    return pl.pallas_call(
        body, grid=(GRID,),
        in_specs=[
            _ROW_SPEC, _AGG_SPEC, _DEG_SPEC,
            pl.BlockSpec((1, 2 * D), lambda i: (0, 0)),
            pl.BlockSpec((1, 1), lambda i: (0, 0)),
        ],
        out_specs=pl.BlockSpec((1, 1), lambda i: (0, 0)),
        out_shape=jax.ShapeDtypeStruct((1, 1), jnp.float32),
        scratch_shapes=[
            pltpu.VMEM((1, D), jnp.float32),
            pltpu.VMEM((1, D), jnp.float32),
        ],
    )(t2, agg, deg, wp, bp)


def kernel(x, edge_index, W1s, W1n, b1, W2s, W2n, b2, Wp, bp):
    src = edge_index[0]
    dst = edge_index[1]
    pad_e = EP - E
    srcp = jnp.concatenate([src, jnp.zeros((pad_e,), jnp.int32)])
    srcp = srcp.reshape(EP // CHUNK, CHUNK)
    dstp = jnp.concatenate([dst, jnp.full((pad_e,), N, jnp.int32)])
    dstp = dstp.reshape(EP // CHUNK, CHUNK)
    xp = jnp.pad(x, ((0, NP - N), (0, 0)))
    z2 = jnp.zeros((ROWS_PER_TILE, H), jnp.float32)
    z1 = jnp.zeros((ROWS_PER_TILE,), jnp.float32)
    ones = jnp.ones((CHUNK,), jnp.float32)
    b1r = b1.reshape(1, D)
    b2r = b2.reshape(1, D)
    wpr = Wp.reshape(1, 2 * D)
    bpr = bp.reshape(1, 1)

    m1t = _tc_messages(xp, W1n)
    agg1, deg = _sc_agg_deg(srcp, dstp, m1t, z2, z1, ones)
    degr = deg.reshape(2, NP, 1)
    t1 = _tc_self(xp, W1s, b1r)
    h1, m2t = _tc_combine(t1, agg1, degr, W2n)
    (agg2,) = _sc_agg(srcp, dstp, m2t, z2)
    t2 = _tc_self(h1, W2s, b2r)
    out = _tc_finish(t2, agg2, degr, wpr, bpr)
    return out.reshape(1)


# speedup vs baseline: 1.2515x; 1.2515x over previous
import jax
import jax.numpy as jnp
from jax import lax
from jax.experimental import pallas as pl
from jax.experimental.pallas import tpu as pltpu
from jax.experimental.pallas import tpu_sc as plsc

N = 50000
E = 800000
D = 64
H = 32
NTILES = 16
NP = 50176
EP = 819200
CHUNK = 128
IB = 25
R = 5
G = 4
ROWS_PER_TILE = NP // NTILES
E_PER_TILE = EP // NTILES
N_CHUNKS = E_PER_TILE // CHUNK
N_BLOCKS = N_CHUNKS // IB
E_PER_TILE_DEG = EP // (2 * NTILES)
N_CHUNKS_DEG = E_PER_TILE_DEG // CHUNK
N_BLOCKS_DEG = N_CHUNKS_DEG // IB
B = 1024
GRID = NP // B


def _sc_aggregate(do_deg):
    mesh = plsc.VectorSubcoreMesh(core_axis_name="c", subcore_axis_name="s")

    out_type = [jax.ShapeDtypeStruct((2, NP, H), jnp.float32)]
    scratch = [
        pltpu.VMEM((IB * CHUNK,), jnp.int32),
        pltpu.VMEM((IB, CHUNK), jnp.int32),
        pltpu.VMEM((R, CHUNK, H), jnp.float32),
        pltpu.VMEM_SHARED((NP, H), jnp.float32),
        pltpu.SemaphoreType.DMA,
        pltpu.SemaphoreType.DMA,
    ]
    if do_deg:
        out_type.append(jax.ShapeDtypeStruct((2, NP), jnp.float32))
        scratch += [
            pltpu.VMEM((CHUNK,), jnp.float32),
            pltpu.VMEM_SHARED((NP,), jnp.float32),
        ]

    def agg_loop(c, s, src2_hbm, dst_hbm, m_hbm, srcb, dstb, rows, acc,
                 sem_g, sem_s):
        ch0 = s * N_CHUNKS

        def gather(j, slot):
            return pltpu.async_copy(
                m_hbm.at[srcb.at[pl.ds(j * CHUNK, CHUNK)]],
                rows.at[slot], sem_g)

        def block(b, carry):
            blk = ch0 + b * IB
            pltpu.sync_copy(src2_hbm.at[c, pl.ds(blk * CHUNK, IB * CHUNK)],
                            srcb)
            pltpu.sync_copy(dst_hbm.at[pl.ds(blk, IB)], dstb)
            gd = {j: gather(j, j) for j in range(G)}
            sd = {}
            waited = set()
            for j in range(IB):
                gd[j].wait()
                sd[j] = pltpu.async_copy(rows.at[j % R],
                                         acc.at[dstb.at[j]],
                                         sem_s, add=True)
                nj = j + G
                if nj < IB:
                    k = nj - R
                    if k >= 0:
                        sd[k].wait()
                        waited.add(k)
                    gd[nj] = gather(nj, nj % R)
            for j in range(IB):
                if j not in waited:
                    sd[j].wait()
            return carry

        lax.fori_loop(0, N_BLOCKS, block, 0)

    def deg_loop(c, s, dst_hbm, dstb, ones_v, dacc):
        ch0 = (c * NTILES + s) * N_CHUNKS_DEG

        def block(b, carry):
            blk = ch0 + b * IB
            pltpu.sync_copy(dst_hbm.at[pl.ds(blk, IB)], dstb)
            for j in range(IB):
                pltpu.sync_copy(ones_v, dacc.at[dstb.at[j]], add=True)
            return carry

        lax.fori_loop(0, N_BLOCKS_DEG, block, 0)

    def body_deg(src2_hbm, dst_hbm, m_hbm, z2_hbm, z1_hbm, ones_hbm,
                 agg_out, deg_out, srcb, dstb, rows, acc, sem_g, sem_s,
                 ones_v, dacc):
        c = lax.axis_index("c")
        s = lax.axis_index("s")
        r0 = s * ROWS_PER_TILE
        pltpu.sync_copy(z2_hbm, acc.at[pl.ds(r0, ROWS_PER_TILE)])
        pltpu.sync_copy(z1_hbm, dacc.at[pl.ds(r0, ROWS_PER_TILE)])
        pltpu.sync_copy(ones_hbm, ones_v)
        plsc.subcore_barrier()

        agg_loop(c, s, src2_hbm, dst_hbm, m_hbm, srcb, dstb, rows, acc,
                 sem_g, sem_s)
        deg_loop(c, s, dst_hbm, dstb, ones_v, dacc)

        plsc.subcore_barrier()
        pltpu.sync_copy(acc.at[pl.ds(r0, ROWS_PER_TILE)],
                        agg_out.at[c, pl.ds(r0, ROWS_PER_TILE)])
        pltpu.sync_copy(dacc.at[pl.ds(r0, ROWS_PER_TILE)],
                        deg_out.at[c, pl.ds(r0, ROWS_PER_TILE)])

    def body_nodeg(src2_hbm, dst_hbm, m_hbm, z2_hbm,
                   agg_out, srcb, dstb, rows, acc, sem_g, sem_s):
        c = lax.axis_index("c")
        s = lax.axis_index("s")
        r0 = s * ROWS_PER_TILE
        pltpu.sync_copy(z2_hbm, acc.at[pl.ds(r0, ROWS_PER_TILE)])
        plsc.subcore_barrier()

        agg_loop(c, s, src2_hbm, dst_hbm, m_hbm, srcb, dstb, rows, acc,
                 sem_g, sem_s)

        plsc.subcore_barrier()
        pltpu.sync_copy(acc.at[pl.ds(r0, ROWS_PER_TILE)],
                        agg_out.at[c, pl.ds(r0, ROWS_PER_TILE)])

    body = body_deg if do_deg else body_nodeg
    return pl.kernel(body, out_type=out_type, mesh=mesh,
                     scratch_types=scratch,
                     compiler_params=pltpu.CompilerParams(
                         use_tc_tiling_on_sc=False))


_sc_agg_deg = _sc_aggregate(True)
_sc_agg = _sc_aggregate(False)

_W_SPEC = pl.BlockSpec((D, D), lambda i: (0, 0))
_B_SPEC = pl.BlockSpec((1, D), lambda i: (0, 0))
_ROW_SPEC = pl.BlockSpec((B, D), lambda i: (i, 0))
_M_SPEC = pl.BlockSpec((2, B, H), lambda i: (0, i, 0))
_DEG_SPEC = pl.BlockSpec((2, B, 1), lambda i: (0, i, 0))
_M_SHAPE = jax.ShapeDtypeStruct((2, NP, H), jnp.float32)
_T_SHAPE = jax.ShapeDtypeStruct((NP, D), jnp.float32)


def _tc_encode(h, Ws, Wn, b):
    def body(h_ref, ws_ref, wn_ref, b_ref, t_ref, m_ref):
        hb = h_ref[...]
        t_ref[...] = jnp.dot(hb, ws_ref[...],
                             preferred_element_type=jnp.float32) + b_ref[...]
        mm = jnp.dot(hb, wn_ref[...], preferred_element_type=jnp.float32)
        m_ref[0] = mm[:, :H]
        m_ref[1] = mm[:, H:]

    return pl.pallas_call(
        body, grid=(GRID,),
        in_specs=[_ROW_SPEC, _W_SPEC, _W_SPEC, _B_SPEC],
        out_specs=[_ROW_SPEC, _M_SPEC],
        out_shape=[_T_SHAPE, _M_SHAPE],
    )(h, Ws, Wn, b)


def _tc_combine_encode(t1, agg, deg, Ws, Wn, b):
    def body(t_ref, a_ref, d_ref, ws_ref, wn_ref, b_ref, t_out, m_out):
        a = jnp.concatenate([a_ref[0], a_ref[1]], axis=1)
        dg = jnp.maximum(d_ref[0] + d_ref[1], 1.0)
        h1 = jnp.maximum(t_ref[...] + a / dg, 0.0)
        t_out[...] = jnp.dot(h1, ws_ref[...],
                             preferred_element_type=jnp.float32) + b_ref[...]
        mm = jnp.dot(h1, wn_ref[...], preferred_element_type=jnp.float32)
        m_out[0] = mm[:, :H]
        m_out[1] = mm[:, H:]

    return pl.pallas_call(
        body, grid=(GRID,),
        in_specs=[_ROW_SPEC, _M_SPEC, _DEG_SPEC, _W_SPEC, _W_SPEC,
                  _B_SPEC],
        out_specs=[_ROW_SPEC, _M_SPEC],
        out_shape=[_T_SHAPE, _M_SHAPE],
    )(t1, agg, deg, Ws, Wn, b)


def _tc_finish(t2, agg, deg, wp, bp):
    def body(t_ref, a_ref, d_ref, wp_ref, bp_ref, o_ref, mx, sm):
        i = pl.program_id(0)
        a = jnp.concatenate([a_ref[0], a_ref[1]], axis=1)
        dg = jnp.maximum(d_ref[0] + d_ref[1], 1.0)
        h2 = t_ref[...] + a / dg
        rid = i * B + lax.broadcasted_iota(jnp.int32, (B, 1), 0)
        valid = rid < N
        pmax = jnp.max(jnp.where(valid, h2, -jnp.inf), axis=0,
                       keepdims=True)
        psum = jnp.sum(jnp.where(valid, h2, 0.0), axis=0, keepdims=True)

        @pl.when(i == 0)
        def _():
            mx[...] = pmax
            sm[...] = psum

        @pl.when(i > 0)
        def _():
            mx[...] = jnp.maximum(mx[...], pmax)
            sm[...] = sm[...] + psum

        @pl.when(i == GRID - 1)
        def _():
            pooled = jnp.concatenate([mx[...], sm[...]], axis=1)
            v = (jnp.sum(pooled * wp_ref[...], axis=1, keepdims=True)
                 + bp_ref[...])
            o_ref[...] = jnp.maximum(v, 0.0) + jnp.log(
                1.0 + jnp.exp(-jnp.abs(v)))

    return pl.pallas_call(
        body, grid=(GRID,),
        in_specs=[
            _ROW_SPEC, _M_SPEC, _DEG_SPEC,
            pl.BlockSpec((1, 2 * D), lambda i: (0, 0)),
            pl.BlockSpec((1, 1), lambda i: (0, 0)),
        ],
        out_specs=pl.BlockSpec((1, 1), lambda i: (0, 0)),
        out_shape=jax.ShapeDtypeStruct((1, 1), jnp.float32),
        scratch_shapes=[
            pltpu.VMEM((1, D), jnp.float32),
            pltpu.VMEM((1, D), jnp.float32),
        ],
    )(t2, agg, deg, wp, bp)


def kernel(x, edge_index, W1s, W1n, b1, W2s, W2n, b2, Wp, bp):
    src = edge_index[0]
    dst = edge_index[1]
    pad_e = EP - E
    srcp = jnp.concatenate([src, jnp.zeros((pad_e,), jnp.int32)])
    dstp = jnp.concatenate([dst, jnp.full((pad_e,), N, jnp.int32)])
    src2 = jnp.stack([srcp, srcp + NP])
    dstp = dstp.reshape(EP // CHUNK, CHUNK)
    xp = jnp.pad(x, ((0, NP - N), (0, 0)))
    z2 = jnp.zeros((ROWS_PER_TILE, H), jnp.float32)
    z1 = jnp.zeros((ROWS_PER_TILE,), jnp.float32)
    ones = jnp.ones((CHUNK,), jnp.float32)
    b1r = b1.reshape(1, D)
    b2r = b2.reshape(1, D)
    wpr = Wp.reshape(1, 2 * D)
    bpr = bp.reshape(1, 1)

    t1, m1 = _tc_encode(xp, W1s, W1n, b1r)
    agg1, deg = _sc_agg_deg(src2, dstp, m1.reshape(2 * NP, H), z2, z1,
                            ones)
    degr = deg.reshape(2, NP, 1)
    t2, m2 = _tc_combine_encode(t1, agg1, degr, W2s, W2n, b2r)
    (agg2,) = _sc_agg(src2, dstp, m2.reshape(2 * NP, H), z2)
    out = _tc_finish(t2, agg2, degr, wpr, bpr)
    return out.reshape(1)
